# Initial kernel scaffold; baseline (speedup 1.0000x reference)
import jax
import jax.numpy as jnp
from jax.experimental import pallas as pl


def kernel(input, table, W_ih, W_hh, b_ih, b_hh):
    raise NotImplementedError("write your pallas kernel here")


# speedup vs baseline: 2.0406x; 2.0406x over previous
import functools

import jax
import jax.numpy as jnp
from jax import lax
from jax.experimental import pallas as pl
from jax.experimental.pallas import tpu as pltpu
from jax.experimental.pallas import tpu_sc as plsc

V = 100000
E = 128
H = 256
B = 1024
L = 50

_CH = 80
_SUB = 400
_GPB = _SUB // _CH


@functools.partial(jax.jit, static_argnums=(2, 3))
def _sc_gather(table, idx3, n_tokens, d):
    info = plsc.get_sparse_core_info()
    nw = info.num_cores * info.num_subcores
    n_per_w = n_tokens // nw
    n_ch = n_per_w // _CH
    mesh = plsc.VectorSubcoreMesh(core_axis_name="c", subcore_axis_name="s")

    @functools.partial(
        pl.kernel,
        mesh=mesh,
        out_type=jax.ShapeDtypeStruct((n_tokens, d), jnp.float32),
        scratch_types=[
            pltpu.VMEM((idx3.shape[1], _CH), jnp.int32),
            pltpu.VMEM((_CH, d), jnp.float32),
            pltpu.SemaphoreType.DMA,
        ],
    )
    def gather_k(table_hbm, idx_hbm, out_hbm, idx_v, rows, sem):
        wid = lax.axis_index("s") * info.num_cores + lax.axis_index("c")
        base = wid * n_per_w
        pltpu.sync_copy(idx_hbm.at[wid], idx_v)
        for j in range(n_ch):
            pltpu.async_copy(table_hbm.at[idx_v.at[j]], rows, sem).wait()
            pltpu.sync_copy(rows, out_hbm.at[pl.ds(base + j * _CH, _CH)])

    return gather_k(table, idx3)


def _lstm_body(x_ref, wih_ref, whh_ref, b_ref, out_ref, hn_ref, cn_ref, h_scr, c_scr):
    step = pl.program_id(0)

    @pl.when(step == 0)
    def _init():
        h_scr[...] = jnp.zeros_like(h_scr)
        c_scr[...] = jnp.zeros_like(c_scr)

    x = x_ref[...]
    h = h_scr[...]
    gates = (
        jnp.dot(x, wih_ref[...], preferred_element_type=jnp.float32)
        + jnp.dot(h, whh_ref[...], preferred_element_type=jnp.float32)
        + b_ref[...]
    )
    i = jax.nn.sigmoid(gates[:, 0:H])
    f = jax.nn.sigmoid(gates[:, H : 2 * H])
    g = jnp.tanh(gates[:, 2 * H : 3 * H])
    o = jax.nn.sigmoid(gates[:, 3 * H : 4 * H])
    c_new = f * c_scr[...] + i * g
    h_new = o * jnp.tanh(c_new)
    h_scr[...] = h_new
    c_scr[...] = c_new
    out_ref[...] = h_new
    hn_ref[...] = h_new
    cn_ref[...] = c_new


def _lstm(xs2d, wih_t, whh_t, bias):
    return pl.pallas_call(
        _lstm_body,
        grid=(L,),
        in_specs=[
            pl.BlockSpec((B, E), lambda l: (0, l)),
            pl.BlockSpec((E, 4 * H), lambda l: (0, 0)),
            pl.BlockSpec((H, 4 * H), lambda l: (0, 0)),
            pl.BlockSpec((1, 4 * H), lambda l: (0, 0)),
        ],
        out_specs=[
            pl.BlockSpec((B, H), lambda l: (0, l)),
            pl.BlockSpec((B, H), lambda l: (0, 0)),
            pl.BlockSpec((B, H), lambda l: (0, 0)),
        ],
        out_shape=[
            jax.ShapeDtypeStruct((B, L * H), jnp.float32),
            jax.ShapeDtypeStruct((B, H), jnp.float32),
            jax.ShapeDtypeStruct((B, H), jnp.float32),
        ],
        scratch_shapes=[
            pltpu.VMEM((B, H), jnp.float32),
            pltpu.VMEM((B, H), jnp.float32),
        ],
        compiler_params=pltpu.CompilerParams(
            dimension_semantics=("arbitrary",),
        ),
    )(xs2d, wih_t, whh_t, bias)


def kernel(input, table, W_ih, W_hh, b_ih, b_hh):
    n = B * L
    info = plsc.get_sparse_core_info()
    nw = info.num_cores * info.num_subcores
    idx3 = input.astype(jnp.int32).reshape(nw, (n // nw) // _CH, _CH)
    emb = _sc_gather(table, idx3, n, E)
    xs2d = emb.reshape(B, L * E)
    wih_t = W_ih.T
    whh_t = W_hh.T
    bias = (b_ih + b_hh).reshape(1, 4 * H)
    out2d, hn, cn = _lstm(xs2d, wih_t, whh_t, bias)
    out = out2d.reshape(B, L, H)
    return (out, hn[None, :, :], cn[None, :, :])


# speedup vs baseline: 2.2656x; 1.1103x over previous
import functools

import jax
import jax.numpy as jnp
from jax import lax
from jax.experimental import pallas as pl
from jax.experimental.pallas import tpu as pltpu
from jax.experimental.pallas import tpu_sc as plsc

V = 100000
E = 128
H = 256
B = 1024
L = 50

_CH = 80
_SUB = 400
_GPB = _SUB // _CH


@functools.partial(jax.jit, static_argnums=(2, 3))
def _sc_gather(table, idx3, n_tokens, d):
    info = plsc.get_sparse_core_info()
    nw = info.num_cores * info.num_subcores
    n_per_w = n_tokens // nw
    n_ch = n_per_w // _CH
    mesh = plsc.VectorSubcoreMesh(core_axis_name="c", subcore_axis_name="s")

    @functools.partial(
        pl.kernel,
        mesh=mesh,
        out_type=jax.ShapeDtypeStruct((n_tokens, d), jnp.float32),
        scratch_types=[
            pltpu.VMEM((idx3.shape[1], _CH), jnp.int32),
            pltpu.VMEM((_CH, d), jnp.float32),
            pltpu.SemaphoreType.DMA,
        ],
    )
    def gather_k(table_hbm, idx_hbm, out_hbm, idx_v, rows, sem):
        wid = lax.axis_index("s") * info.num_cores + lax.axis_index("c")
        base = wid * n_per_w
        pltpu.sync_copy(idx_hbm.at[wid], idx_v)
        for j in range(n_ch):
            pltpu.async_copy(table_hbm.at[idx_v.at[j]], rows, sem).wait()
            pltpu.sync_copy(rows, out_hbm.at[pl.ds(base + j * _CH, _CH)])

    return gather_k(table, idx3)


def _lstm_body(x_ref, wih_ref, whh_ref, b_ref, out_ref, hn_ref, cn_ref, h_scr, c_scr):
    step = pl.program_id(0)

    @pl.when(step == 0)
    def _init():
        h_scr[...] = jnp.zeros_like(h_scr)
        c_scr[...] = jnp.zeros_like(c_scr)

    x = x_ref[...]
    h = h_scr[...]
    gates = (
        jnp.dot(x, wih_ref[...], preferred_element_type=jnp.float32)
        + jnp.dot(h, whh_ref[...], preferred_element_type=jnp.float32)
        + b_ref[...]
    )
    i = jax.nn.sigmoid(gates[:, 0:H])
    f = jax.nn.sigmoid(gates[:, H : 2 * H])
    g = jnp.tanh(gates[:, 2 * H : 3 * H])
    o = jax.nn.sigmoid(gates[:, 3 * H : 4 * H])
    c_new = f * c_scr[...] + i * g
    h_new = o * jnp.tanh(c_new)
    h_scr[...] = h_new
    c_scr[...] = c_new
    out_ref[...] = h_new
    hn_ref[...] = h_new
    cn_ref[...] = c_new


def _lstm(xs_lm, wih_t, whh_t, bias):
    return pl.pallas_call(
        _lstm_body,
        grid=(L,),
        in_specs=[
            pl.BlockSpec((B, E), lambda l: (l, 0)),
            pl.BlockSpec((E, 4 * H), lambda l: (0, 0)),
            pl.BlockSpec((H, 4 * H), lambda l: (0, 0)),
            pl.BlockSpec((1, 4 * H), lambda l: (0, 0)),
        ],
        out_specs=[
            pl.BlockSpec((B, H), lambda l: (0, l)),
            pl.BlockSpec((B, H), lambda l: (0, 0)),
            pl.BlockSpec((B, H), lambda l: (0, 0)),
        ],
        out_shape=[
            jax.ShapeDtypeStruct((B, L * H), jnp.float32),
            jax.ShapeDtypeStruct((B, H), jnp.float32),
            jax.ShapeDtypeStruct((B, H), jnp.float32),
        ],
        scratch_shapes=[
            pltpu.VMEM((B, H), jnp.float32),
            pltpu.VMEM((B, H), jnp.float32),
        ],
        compiler_params=pltpu.CompilerParams(
            dimension_semantics=("arbitrary",),
        ),
    )(xs_lm, wih_t, whh_t, bias)


def kernel(input, table, W_ih, W_hh, b_ih, b_hh):
    n = B * L
    info = plsc.get_sparse_core_info()
    nw = info.num_cores * info.num_subcores
    idx3 = input.astype(jnp.int32).T.reshape(nw, (n // nw) // _CH, _CH)
    emb = _sc_gather(table, idx3, n, E)
    wih_t = W_ih.T
    whh_t = W_hh.T
    bias = (b_ih + b_hh).reshape(1, 4 * H)
    out2d, hn, cn = _lstm(emb, wih_t, whh_t, bias)
    out = out2d.reshape(B, L, H)
    return (out, hn[None, :, :], cn[None, :, :])
